# baseline (device time: 166150 ns/iter reference)
import functools
import os

import jax
import jax.numpy as jnp
from jax import lax
from jax.experimental import pallas as pl
from jax.experimental.pallas import tpu as pltpu

N_DEV = 4
SQ = 2048
SKV = 2048
D_MODEL = 1024
H_PER = 8
DH = 128
QBLK = 256
LSEG = 512
GSEG = 128
L_OFF = N_DEV * QBLK
SCALE = 0.08838834764831843
NEG = -1e9

_SKIP_COMM = os.environ.get("K_SKIP_COMM") == "1"
_SKIP_COMPUTE = os.environ.get("K_SKIP_COMPUTE") == "1"


def _body(x_ref, wq_ref, k_ref, v_ref, wo_ref, out_ref,
          qblk_ref, ctx_ref,
          rbR, rbL, sbR, sbL, agsR, agsL, agrR, agrL,
          sendR, recvR, sendL, recvL,
          sendAgR, recvAgR, sendAgL, recvAgL):
    my = lax.axis_index("i")
    right = lax.rem(my + 1, N_DEV)
    left = lax.rem(my + N_DEV - 1, N_DEV)

    barrier = pltpu.get_barrier_semaphore()
    for nbr in (left, right):
        pl.semaphore_signal(barrier, inc=1, device_id=(nbr,),
                            device_id_type=pl.DeviceIdType.MESH)
    pl.semaphore_wait(barrier, 2)

    def compute_block(g):
        if _SKIP_COMPUTE:
            return
        row0 = pl.multiple_of(g * QBLK, QBLK)
        qblk_ref[:] = (jnp.dot(
            x_ref[pl.ds(row0, QBLK), :], wq_ref[:],
            preferred_element_type=jnp.float32) * SCALE).astype(jnp.bfloat16)

        seg = jnp.clip(g * QBLK - 128, 0, SKV - LSEG)
        delta = g * QBLK - seg
        seg = pl.multiple_of(seg, 128)
        r = lax.broadcasted_iota(jnp.int32, (QBLK, LSEG), 0)
        j = lax.broadcasted_iota(jnp.int32, (QBLK, LSEG), 1)
        is_b0 = g == 0
        band = jnp.abs(delta + r - j) <= 128
        mask_l = band | (is_b0 & (j < 32))
        pen_l = jnp.where(mask_l, 0.0, NEG).astype(jnp.float32)
        jg = lax.broadcasted_iota(jnp.int32, (QBLK, GSEG), 1)
        pen_g = jnp.where(jnp.logical_not(is_b0) & (jg < 32), 0.0,
                          NEG).astype(jnp.float32)

        for h in range(H_PER):
            hc = slice(h * DH, (h + 1) * DH)
            qh = qblk_ref[:, hc]
            kl = k_ref[pl.ds(seg, LSEG), hc]
            s_l = lax.dot_general(qh, kl, (((1,), (1,)), ((), ())),
                                  preferred_element_type=jnp.float32) + pen_l
            kg = k_ref[0:GSEG, hc]
            s_g = lax.dot_general(qh, kg, (((1,), (1,)), ((), ())),
                                  preferred_element_type=jnp.float32) + pen_g
            wl = jnp.exp(s_l)
            wg = jnp.exp(s_g)
            den = (jnp.sum(wl, axis=1, keepdims=True)
                   + jnp.sum(wg, axis=1, keepdims=True))
            ctx = (jnp.dot(wl.astype(jnp.bfloat16), v_ref[pl.ds(seg, LSEG), hc],
                           preferred_element_type=jnp.float32)
                   + jnp.dot(wg.astype(jnp.bfloat16), v_ref[0:GSEG, hc],
                             preferred_element_type=jnp.float32)) / den
            ctx_ref[:, hc] = ctx.astype(jnp.bfloat16)

        @pl.when(g == 0)
        def _():
            for h in range(H_PER):
                hc = slice(h * DH, (h + 1) * DH)
                qh = qblk_ref[0:32, hc]
                s = lax.dot_general(qh, k_ref[:, hc], (((1,), (1,)), ((), ())),
                                    preferred_element_type=jnp.float32)
                w = jnp.exp(s)
                ctx = jnp.dot(w.astype(jnp.bfloat16), v_ref[:, hc],
                              preferred_element_type=jnp.float32)
                ctx = ctx / jnp.sum(w, axis=1, keepdims=True)
                ctx_ref[0:32, hc] = ctx.astype(jnp.bfloat16)

        out_ref[pl.ds(row0, QBLK), :] = jnp.dot(
            ctx_ref[:], wo_ref[:], preferred_element_type=jnp.float32)

    def r_rows(c):
        return pl.ds(pl.multiple_of(c * QBLK, QBLK), QBLK)

    def l_rows(c):
        return pl.ds(pl.multiple_of(L_OFF + c * QBLK, QBLK), QBLK)

    if _SKIP_COMM:
        for g in range(2 * N_DEV):
            compute_block(g)
        return

    compute_block(my)
    compute_block(N_DEV + my)
    sbR[0] = out_ref[r_rows(my), :].astype(jnp.bfloat16)
    sbL[0] = out_ref[l_rows(my), :].astype(jnp.bfloat16)
    rR = pltpu.make_async_remote_copy(
        src_ref=sbR.at[0], dst_ref=rbR.at[0],
        send_sem=sendR.at[0], recv_sem=recvR.at[0],
        device_id=(right,), device_id_type=pl.DeviceIdType.MESH)
    rL = pltpu.make_async_remote_copy(
        src_ref=sbL.at[0], dst_ref=rbL.at[0],
        send_sem=sendL.at[0], recv_sem=recvL.at[0],
        device_id=(left,), device_id_type=pl.DeviceIdType.MESH)
    rR.start()
    rL.start()

    for j in range(1, N_DEV):
        cR = lax.rem(my + N_DEV - j, N_DEV)
        cL = lax.rem(my + j, N_DEV)
        compute_block(cR)
        compute_block(N_DEV + cL)
        rR.wait()
        rL.wait()
        accR = out_ref[r_rows(cR), :] + rbR[j - 1].astype(jnp.float32)
        accL = out_ref[l_rows(cL), :] + rbL[j - 1].astype(jnp.float32)
        out_ref[r_rows(cR), :] = accR
        out_ref[l_rows(cL), :] = accL
        if j <= N_DEV - 2:
            sbR[j] = accR.astype(jnp.bfloat16)
            sbL[j] = accL.astype(jnp.bfloat16)
            rR = pltpu.make_async_remote_copy(
                src_ref=sbR.at[j], dst_ref=rbR.at[j],
                send_sem=sendR.at[j], recv_sem=recvR.at[j],
                device_id=(right,), device_id_type=pl.DeviceIdType.MESH)
            rL = pltpu.make_async_remote_copy(
                src_ref=sbL.at[j], dst_ref=rbL.at[j],
                send_sem=sendL.at[j], recv_sem=recvL.at[j],
                device_id=(left,), device_id_type=pl.DeviceIdType.MESH)
            rR.start()
            rL.start()

    agsR[:] = out_ref[r_rows(lax.rem(my + 1, N_DEV)), :].astype(jnp.bfloat16)
    agsL[:] = out_ref[l_rows(lax.rem(my + N_DEV - 1, N_DEV)), :].astype(
        jnp.bfloat16)
    for h in range(N_DEV - 1):
        srcR = agsR if h == 0 else agrR.at[h - 1]
        srcL = agsL if h == 0 else agrL.at[h - 1]
        rR = pltpu.make_async_remote_copy(
            src_ref=srcR, dst_ref=agrR.at[h],
            send_sem=sendAgR.at[h], recv_sem=recvAgR.at[h],
            device_id=(right,), device_id_type=pl.DeviceIdType.MESH)
        rL = pltpu.make_async_remote_copy(
            src_ref=srcL, dst_ref=agrL.at[h],
            send_sem=sendAgL.at[h], recv_sem=recvAgL.at[h],
            device_id=(left,), device_id_type=pl.DeviceIdType.MESH)
        rR.start()
        rL.start()
        if h >= 1:
            cRp = lax.rem(my + N_DEV - (h - 1), N_DEV)
            cLp = lax.rem(my + (h - 1), N_DEV)
            out_ref[r_rows(cRp), :] = agrR[h - 1].astype(jnp.float32)
            out_ref[l_rows(cLp), :] = agrL[h - 1].astype(jnp.float32)
        rR.wait()
        rL.wait()
    h_last = N_DEV - 2
    out_ref[r_rows(lax.rem(my + N_DEV - h_last, N_DEV)), :] = (
        agrR[h_last].astype(jnp.float32))
    out_ref[l_rows(lax.rem(my + h_last, N_DEV)), :] = (
        agrL[h_last].astype(jnp.float32))

    @functools.partial(pl.run_scoped, sem=pltpu.SemaphoreType.REGULAR)
    def _(sem):
        for nbr in (left, right):
            pl.semaphore_signal(sem, inc=1, device_id=(nbr,),
                                device_id_type=pl.DeviceIdType.MESH)
        pl.semaphore_wait(sem, 2)


def kernel(x, Wq, K_ext, V_ext, Wo):
    i = lax.axis_index("i")
    K2 = K_ext.reshape(SKV, N_DEV * H_PER * DH)
    V2 = V_ext.reshape(SKV, N_DEV * H_PER * DH)
    K = lax.dynamic_slice_in_dim(
        K2, i * H_PER * DH, H_PER * DH, axis=1).astype(jnp.bfloat16)
    V = lax.dynamic_slice_in_dim(
        V2, i * H_PER * DH, H_PER * DH, axis=1).astype(jnp.bfloat16)
    x16 = x[0].astype(jnp.bfloat16)
    Wq16 = Wq.astype(jnp.bfloat16)
    Wo16 = Wo.astype(jnp.bfloat16)

    dma3 = pltpu.SemaphoreType.DMA((N_DEV - 1,))
    buf3 = pltpu.VMEM((N_DEV - 1, QBLK, D_MODEL), jnp.bfloat16)
    buf1 = pltpu.VMEM((QBLK, D_MODEL), jnp.bfloat16)
    out = pl.pallas_call(
        _body,
        out_shape=jax.ShapeDtypeStruct((SQ, D_MODEL), jnp.float32),
        in_specs=[pl.BlockSpec(memory_space=pltpu.VMEM)] * 5,
        out_specs=pl.BlockSpec(memory_space=pltpu.VMEM),
        scratch_shapes=[
            pltpu.VMEM((QBLK, H_PER * DH), jnp.bfloat16),
            pltpu.VMEM((QBLK, H_PER * DH), jnp.bfloat16),
            buf3, buf3,
            buf3, buf3,
            buf1, buf1,
            buf3, buf3,
            dma3, dma3,
            dma3, dma3,
            dma3, dma3,
            dma3, dma3,
        ],
        compiler_params=pltpu.CompilerParams(
            collective_id=0, vmem_limit_bytes=128 * 1024 * 1024),
    )(x16, Wq16, K, V, Wo16)
    return out[None]


# device time: 96425 ns/iter; 1.7231x vs baseline; 1.7231x over previous
import functools
import os

import jax
import jax.numpy as jnp
from jax import lax
from jax.experimental import pallas as pl
from jax.experimental.pallas import tpu as pltpu

N_DEV = 4
SQ = 2048
SKV = 2048
D_MODEL = 1024
H_PER = 8
DH = 128
QBLK = 256
LSEG = 512
GSEG = 128
L_OFF = N_DEV * QBLK
SCALE = 0.08838834764831843
NEG = -1e9

_SKIP_COMM = os.environ.get("K_SKIP_COMM") == "1"
_SKIP_COMPUTE = os.environ.get("K_SKIP_COMPUTE") == "1"


def _body(x_ref, wq_ref, k_ref, v_ref, wo_ref, out_ref,
          qblk_ref, ctx_ref,
          rbR, rbL, sbR, sbL, agsR, agsL, agrR, agrL,
          sendR, recvR, sendL, recvL,
          sendAgR, recvAgR, sendAgL, recvAgL):
    my = lax.axis_index("i")
    right = lax.rem(my + 1, N_DEV)
    left = lax.rem(my + N_DEV - 1, N_DEV)

    barrier = pltpu.get_barrier_semaphore()
    for nbr in (left, right):
        pl.semaphore_signal(barrier, inc=1, device_id=(nbr,),
                            device_id_type=pl.DeviceIdType.MESH)
    pl.semaphore_wait(barrier, 2)

    def compute_block(g):
        if _SKIP_COMPUTE:
            return
        row0 = pl.multiple_of(g * QBLK, QBLK)
        qblk_ref[:] = (jnp.dot(
            x_ref[pl.ds(row0, QBLK), :], wq_ref[:],
            preferred_element_type=jnp.float32) * SCALE).astype(jnp.bfloat16)

        seg = jnp.clip(g * QBLK - 128, 0, SKV - LSEG)
        delta = g * QBLK - seg
        seg = pl.multiple_of(seg, 128)
        r = lax.broadcasted_iota(jnp.int32, (QBLK, LSEG), 0)
        j = lax.broadcasted_iota(jnp.int32, (QBLK, LSEG), 1)
        is_b0 = g == 0
        band = jnp.abs(delta + r - j) <= 128
        mask_l = band | (is_b0 & (j < 32))
        pen_l = jnp.where(mask_l, 0.0, NEG).astype(jnp.float32)
        jg = lax.broadcasted_iota(jnp.int32, (QBLK, GSEG), 1)
        pen_g = jnp.where(jnp.logical_not(is_b0) & (jg < 32), 0.0,
                          NEG).astype(jnp.float32)

        for h in range(H_PER):
            hc = slice(h * DH, (h + 1) * DH)
            qh = qblk_ref[:, hc]
            kl = k_ref[h, pl.ds(seg, LSEG), :]
            s_l = lax.dot_general(qh, kl, (((1,), (1,)), ((), ())),
                                  preferred_element_type=jnp.float32) + pen_l
            kg = k_ref[h, 0:GSEG, :]
            s_g = lax.dot_general(qh, kg, (((1,), (1,)), ((), ())),
                                  preferred_element_type=jnp.float32) + pen_g
            wl = jnp.exp(s_l)
            wg = jnp.exp(s_g)
            den = (jnp.sum(wl, axis=1, keepdims=True)
                   + jnp.sum(wg, axis=1, keepdims=True))
            ctx = (jnp.dot(wl.astype(jnp.bfloat16),
                           v_ref[h, pl.ds(seg, LSEG), :],
                           preferred_element_type=jnp.float32)
                   + jnp.dot(wg.astype(jnp.bfloat16), v_ref[h, 0:GSEG, :],
                             preferred_element_type=jnp.float32)) / den
            ctx_ref[:, hc] = ctx

        @pl.when(g == 0)
        def _():
            for h in range(H_PER):
                hc = slice(h * DH, (h + 1) * DH)
                qh = qblk_ref[0:32, hc]
                s = lax.dot_general(qh, k_ref[h], (((1,), (1,)), ((), ())),
                                    preferred_element_type=jnp.float32)
                w = jnp.exp(s)
                ctx = jnp.dot(w.astype(jnp.bfloat16), v_ref[h],
                              preferred_element_type=jnp.float32)
                ctx = ctx / jnp.sum(w, axis=1, keepdims=True)
                ctx_ref[0:32, hc] = ctx

        out_ref[pl.ds(row0, QBLK), :] = jnp.dot(
            ctx_ref[:], wo_ref[:], preferred_element_type=jnp.float32)

    def r_rows(c):
        return pl.ds(pl.multiple_of(c * QBLK, QBLK), QBLK)

    def l_rows(c):
        return pl.ds(pl.multiple_of(L_OFF + c * QBLK, QBLK), QBLK)

    if _SKIP_COMM:
        for g in range(2 * N_DEV):
            compute_block(g)
        return

    compute_block(my)
    compute_block(N_DEV + my)
    sbR[0] = out_ref[r_rows(my), :].astype(jnp.bfloat16)
    sbL[0] = out_ref[l_rows(my), :].astype(jnp.bfloat16)
    rR = pltpu.make_async_remote_copy(
        src_ref=sbR.at[0], dst_ref=rbR.at[0],
        send_sem=sendR.at[0], recv_sem=recvR.at[0],
        device_id=(right,), device_id_type=pl.DeviceIdType.MESH)
    rL = pltpu.make_async_remote_copy(
        src_ref=sbL.at[0], dst_ref=rbL.at[0],
        send_sem=sendL.at[0], recv_sem=recvL.at[0],
        device_id=(left,), device_id_type=pl.DeviceIdType.MESH)
    rR.start()
    rL.start()

    for j in range(1, N_DEV):
        cR = lax.rem(my + N_DEV - j, N_DEV)
        cL = lax.rem(my + j, N_DEV)
        compute_block(cR)
        compute_block(N_DEV + cL)
        rR.wait()
        rL.wait()
        accR = out_ref[r_rows(cR), :] + rbR[j - 1].astype(jnp.float32)
        accL = out_ref[l_rows(cL), :] + rbL[j - 1].astype(jnp.float32)
        out_ref[r_rows(cR), :] = accR
        out_ref[l_rows(cL), :] = accL
        if j <= N_DEV - 2:
            sbR[j] = accR.astype(jnp.bfloat16)
            sbL[j] = accL.astype(jnp.bfloat16)
            rR = pltpu.make_async_remote_copy(
                src_ref=sbR.at[j], dst_ref=rbR.at[j],
                send_sem=sendR.at[j], recv_sem=recvR.at[j],
                device_id=(right,), device_id_type=pl.DeviceIdType.MESH)
            rL = pltpu.make_async_remote_copy(
                src_ref=sbL.at[j], dst_ref=rbL.at[j],
                send_sem=sendL.at[j], recv_sem=recvL.at[j],
                device_id=(left,), device_id_type=pl.DeviceIdType.MESH)
            rR.start()
            rL.start()

    agsR[:] = out_ref[r_rows(lax.rem(my + 1, N_DEV)), :].astype(jnp.bfloat16)
    agsL[:] = out_ref[l_rows(lax.rem(my + N_DEV - 1, N_DEV)), :].astype(
        jnp.bfloat16)
    for h in range(N_DEV - 1):
        srcR = agsR if h == 0 else agrR.at[h - 1]
        srcL = agsL if h == 0 else agrL.at[h - 1]
        rR = pltpu.make_async_remote_copy(
            src_ref=srcR, dst_ref=agrR.at[h],
            send_sem=sendAgR.at[h], recv_sem=recvAgR.at[h],
            device_id=(right,), device_id_type=pl.DeviceIdType.MESH)
        rL = pltpu.make_async_remote_copy(
            src_ref=srcL, dst_ref=agrL.at[h],
            send_sem=sendAgL.at[h], recv_sem=recvAgL.at[h],
            device_id=(left,), device_id_type=pl.DeviceIdType.MESH)
        rR.start()
        rL.start()
        if h >= 1:
            cRp = lax.rem(my + N_DEV - (h - 1), N_DEV)
            cLp = lax.rem(my + (h - 1), N_DEV)
            out_ref[r_rows(cRp), :] = agrR[h - 1].astype(jnp.float32)
            out_ref[l_rows(cLp), :] = agrL[h - 1].astype(jnp.float32)
        rR.wait()
        rL.wait()
    h_last = N_DEV - 2
    out_ref[r_rows(lax.rem(my + N_DEV - h_last, N_DEV)), :] = (
        agrR[h_last].astype(jnp.float32))
    out_ref[l_rows(lax.rem(my + h_last, N_DEV)), :] = (
        agrL[h_last].astype(jnp.float32))

    @functools.partial(pl.run_scoped, sem=pltpu.SemaphoreType.REGULAR)
    def _(sem):
        for nbr in (left, right):
            pl.semaphore_signal(sem, inc=1, device_id=(nbr,),
                                device_id_type=pl.DeviceIdType.MESH)
        pl.semaphore_wait(sem, 2)


def kernel(x, Wq, K_ext, V_ext, Wo):
    i = lax.axis_index("i")
    K = lax.dynamic_slice_in_dim(K_ext[0], i * H_PER, H_PER, axis=1)
    V = lax.dynamic_slice_in_dim(V_ext[0], i * H_PER, H_PER, axis=1)
    K = jnp.transpose(K, (1, 0, 2)).astype(jnp.bfloat16)
    V = jnp.transpose(V, (1, 0, 2)).astype(jnp.bfloat16)

    dma3 = pltpu.SemaphoreType.DMA((N_DEV - 1,))
    buf3 = pltpu.VMEM((N_DEV - 1, QBLK, D_MODEL), jnp.bfloat16)
    buf1 = pltpu.VMEM((QBLK, D_MODEL), jnp.bfloat16)
    out = pl.pallas_call(
        _body,
        out_shape=jax.ShapeDtypeStruct((SQ, D_MODEL), jnp.float32),
        in_specs=[pl.BlockSpec(memory_space=pltpu.VMEM)] * 5,
        out_specs=pl.BlockSpec(memory_space=pltpu.VMEM),
        scratch_shapes=[
            pltpu.VMEM((QBLK, H_PER * DH), jnp.bfloat16),
            pltpu.VMEM((QBLK, H_PER * DH), jnp.float32),
            buf3, buf3,
            buf3, buf3,
            buf1, buf1,
            buf3, buf3,
            dma3, dma3,
            dma3, dma3,
            dma3, dma3,
            dma3, dma3,
        ],
        compiler_params=pltpu.CompilerParams(
            collective_id=0, vmem_limit_bytes=128 * 1024 * 1024),
    )(x[0], Wq, K, V, Wo)
    return out[None]
